# Initial kernel scaffold; baseline (speedup 1.0000x reference)
import jax
import jax.numpy as jnp
from jax.experimental import pallas as pl


def kernel(positions, sizes, densities, colors, ray_origins, ray_directions):
    raise NotImplementedError("write your pallas kernel here")


# speedup vs baseline: 7.8537x; 7.8537x over previous
import functools

import jax
import jax.numpy as jnp
from jax import lax
from jax.experimental import pallas as pl
from jax.experimental.pallas import tpu as pltpu
from jax.experimental.pallas import tpu_sc as plsc

N = 4096
B = 1024
RB = 128
NW = 32
RPW = B // NW
K = 64
KPAD = 96
NG = K // 16
BIG = 1e30
BIGTH = 1e29
FAR = 100.0


def _tc_body(ro_ref, inv_ref, bmin_ref, bmax_ref, key_ref, tfar_ref):
    tn = None
    tf = None
    for a in range(3):
        o = ro_ref[:, a:a + 1]
        iv = inv_ref[:, a:a + 1]
        lo = bmin_ref[a:a + 1, :]
        hi = bmax_ref[a:a + 1, :]
        ta = (lo - o) * iv
        tb = (hi - o) * iv
        t1 = jnp.minimum(ta, tb)
        t2 = jnp.maximum(ta, tb)
        tn = t1 if tn is None else jnp.maximum(tn, t1)
        tf = t2 if tf is None else jnp.minimum(tf, t2)
    valid = (tf > tn) & (tf > 0.0)
    key_ref[...] = jnp.where(valid, tn, BIG)
    tfar_ref[...] = tf


def _tc_intersect(ro, inv, bmin_t, bmax_t):
    return pl.pallas_call(
        _tc_body,
        grid=(B // RB,),
        in_specs=[
            pl.BlockSpec((RB, 3), lambda i: (i, 0)),
            pl.BlockSpec((RB, 3), lambda i: (i, 0)),
            pl.BlockSpec((3, N), lambda i: (0, 0)),
            pl.BlockSpec((3, N), lambda i: (0, 0)),
        ],
        out_specs=[
            pl.BlockSpec((RB, N), lambda i: (i, 0)),
            pl.BlockSpec((RB, N), lambda i: (i, 0)),
        ],
        out_shape=[
            jax.ShapeDtypeStruct((B, N), jnp.float32),
            jax.ShapeDtypeStruct((B, N), jnp.float32),
        ],
    )(ro, inv, bmin_t, bmax_t)


def _sc_composite(key, tfar, dens, colors, sh_pad):
    mesh = plsc.VectorSubcoreMesh(core_axis_name="c", subcore_axis_name="s")

    @functools.partial(
        pl.kernel,
        mesh=mesh,
        compiler_params=pltpu.CompilerParams(
            use_tc_tiling_on_sc=False, needs_layout_passes=False),
        out_type=[
            jax.ShapeDtypeStruct((B * 3,), jnp.float32),
            jax.ShapeDtypeStruct((B,), jnp.float32),
            jax.ShapeDtypeStruct((B,), jnp.float32),
        ],
        scratch_types=[
            pltpu.VMEM((N,), jnp.float32),
            pltpu.VMEM((N,), jnp.float32),
            pltpu.VMEM((N,), jnp.float32),
            pltpu.VMEM((16,), jnp.float32),
            pltpu.VMEM((KPAD,), jnp.float32),
            pltpu.VMEM((KPAD,), jnp.float32),
            pltpu.VMEM((KPAD,), jnp.int32),
            pltpu.VMEM((K,), jnp.float32),
            pltpu.VMEM((K,), jnp.float32),
            pltpu.VMEM((N * 27,), jnp.float32),
            pltpu.VMEM((RPW * 16,), jnp.float32),
            pltpu.VMEM((RPW * 16,), jnp.float32),
            pltpu.VMEM((RPW * 16,), jnp.float32),
            pltpu.VMEM((RPW * 16,), jnp.float32),
            pltpu.VMEM((RPW * 16,), jnp.float32),
            pltpu.VMEM((RPW * 16,), jnp.float32),
            pltpu.VMEM((RPW * 3,), jnp.float32),
            pltpu.VMEM((RPW,), jnp.float32),
            pltpu.VMEM((RPW,), jnp.float32),
            pltpu.SemaphoreType.DMA,
        ],
    )
    def sc_kernel(key_hbm, tfar_hbm, dens_hbm, colors_hbm, sh_hbm,
                  rgb_hbm, dep_hbm, wei_hbm,
                  keyrow, tfrow, densv, shrow, hk, htf, hidx, hs, htm,
                  colflat, stR, stG, stB, stD, stW, stS,
                  outrgb, outd, outw, sem):
        wid = lax.axis_index("s") * 2 + lax.axis_index("c")
        base = wid * RPW
        lanes = lax.iota(jnp.int32, 16)
        zeros16 = jnp.zeros((16,), jnp.float32)

        pltpu.sync_copy(dens_hbm, densv)
        pltpu.sync_copy(colors_hbm, colflat)

        def per_ray(r, _):
            ray = base + r
            pltpu.sync_copy(key_hbm.at[ray], keyrow)
            pltpu.sync_copy(tfar_hbm.at[ray], tfrow)
            pltpu.sync_copy(sh_hbm.at[ray], shrow)
            for t in range(KPAD // 16):
                sl = pl.ds(t * 16, 16)
                hk[sl] = jnp.full((16,), BIG, jnp.float32)
                htf[sl] = zeros16
                hidx[sl] = jnp.zeros((16,), jnp.int32)

            def scan_g(g, cnt):
                k16 = keyrow[pl.ds(g * 16, 16)]
                m = k16 < BIGTH
                c = jnp.sum(m.astype(jnp.int32))

                @pl.when(c > 0)
                def _():
                    cc = jnp.minimum(cnt, K)
                    tf16 = tfrow[pl.ds(g * 16, 16)]
                    i16 = g * 16 + lanes
                    plsc.store_compressed(hk.at[pl.ds(cc, 16)], k16, mask=m)
                    plsc.store_compressed(htf.at[pl.ds(cc, 16)], tf16, mask=m)
                    plsc.store_compressed(hidx.at[pl.ds(cc, 16)], i16, mask=m)

                return cnt + c

            cnt = lax.fori_loop(0, N // 16, scan_g, 0)
            H = jnp.minimum(cnt, K)

            for hg in range(NG):
                sl = pl.ds(hg * 16, 16)
                k = hk[sl]
                tfv = htf[sl]
                idx = hidx[sl]
                mval = k < BIGTH
                te = jnp.maximum(k, 0.0)
                sig = jnp.exp(plsc.load_gather(densv, [idx]))
                hs[sl] = jnp.where(mval, sig * (tfv - te), 0.0)
                htm[sl] = jnp.where(mval, 0.5 * (te + tfv), 0.0)

            k_g = [hk[pl.ds(hg * 16, 16)] for hg in range(NG)]
            i_g = [hidx[pl.ds(hg * 16, 16)] for hg in range(NG)]

            def pair_j(j, S):
                jj = jnp.full((16,), j, jnp.int32)
                kj = plsc.load_gather(hk, [jj])
                sj = plsc.load_gather(hs, [jj])
                ij = plsc.load_gather(hidx, [jj])
                out = []
                for hg in range(NG):
                    before = (kj < k_g[hg]) | ((kj == k_g[hg]) & (ij < i_g[hg]))
                    out.append(S[hg] + jnp.where(before, sj, 0.0))
                return tuple(out)

            S = lax.fori_loop(0, H, pair_j,
                              tuple(zeros16 for _ in range(NG)))

            sh_k = [jnp.ones((16,), jnp.float32)] + [
                plsc.load_gather(shrow, [jnp.full((16,), kk, jnp.int32)])
                for kk in range(1, 9)]

            accR = accG = accB = accD = accW = accS = zeros16
            for hg in range(NG):
                sl = pl.ds(hg * 16, 16)
                sv = hs[sl]
                eS = jnp.exp(-S[hg])
                w = eS - eS * jnp.exp(-sv)
                idx27 = hidx[sl] * 27
                cc = []
                for ch in range(3):
                    acc = zeros16
                    for k9 in range(9):
                        col = plsc.load_gather(
                            colflat, [idx27 + (ch * 9 + k9)])
                        acc = acc + sh_k[k9] * col
                    den = 1.0 + jnp.exp(-acc)
                    yv = 1.0 / den
                    yv = yv * (2.0 - den * yv)
                    yv = yv * (2.0 - den * yv)
                    cc.append(yv)
                accR = accR + w * cc[0]
                accG = accG + w * cc[1]
                accB = accB + w * cc[2]
                accD = accD + w * htm[sl]
                accW = accW + w
                accS = accS + sv

            sl = pl.ds(r * 16, 16)
            stR[sl] = accR
            stG[sl] = accG
            stB[sl] = accB
            stD[sl] = accD
            stW[sl] = accW
            stS[sl] = accS
            return 0

        lax.fori_loop(0, RPW, per_ray, 0)

        for rg in range(RPW // 16):
            rows = rg * 16 + lanes

            def colsum(st):
                acc = zeros16
                for l in range(16):
                    acc = acc + plsc.load_gather(st, [rows * 16 + l])
                return acc

            Rv = colsum(stR)
            Gv = colsum(stG)
            Bv = colsum(stB)
            Dv = colsum(stD)
            Wv = colsum(stW)
            Sv = colsum(stS)
            tfin = jnp.exp(-Sv)
            dep = Dv + tfin * FAR
            ridx = rows * 3
            plsc.store_scatter(outrgb, [ridx], Rv)
            plsc.store_scatter(outrgb, [ridx + 1], Gv)
            plsc.store_scatter(outrgb, [ridx + 2], Bv)
            outd[pl.ds(rg * 16, 16)] = dep
            outw[pl.ds(rg * 16, 16)] = Wv

        pltpu.sync_copy(outrgb, rgb_hbm.at[pl.ds(base * 3, RPW * 3)])
        pltpu.sync_copy(outd, dep_hbm.at[pl.ds(base, RPW)])
        pltpu.sync_copy(outw, wei_hbm.at[pl.ds(base, RPW)])

    return sc_kernel(key, tfar, dens, colors, sh_pad)


def kernel(positions, sizes, densities, colors, ray_origins, ray_directions):
    half = (sizes * 0.5)[:, None]
    bmin_t = (positions - half).T
    bmax_t = (positions + half).T
    rdg = jnp.where(jnp.abs(ray_directions) < 1e-8, 1e-8, ray_directions)
    inv = 1.0 / rdg

    x = ray_directions[:, 0]
    y = ray_directions[:, 1]
    z = ray_directions[:, 2]
    sh = jnp.stack([jnp.ones_like(x), y, z, x, x * y, y * z,
                    3.0 * z * z - 1.0, x * z, x * x - y * y], axis=-1)
    sh_pad = jnp.concatenate(
        [sh, jnp.zeros((B, 7), jnp.float32)], axis=1)

    key, tfar = _tc_intersect(ray_origins, inv, bmin_t, bmax_t)

    rgb_flat, dep_flat, wei_flat = _sc_composite(
        key, tfar, densities, colors.reshape(-1), sh_pad)

    return (rgb_flat.reshape(B, 3), dep_flat[:, None], wei_flat[:, None])
